# baseline (speedup 1.0000x reference)
import functools

import jax
import jax.numpy as jnp
from jax import lax
from jax.experimental import pallas as pl
from jax.experimental.pallas import tpu as pltpu
from jax.experimental.pallas import tpu_sc as plsc

N_NODES = 10000
N_EDGES = 320000
D = 128

NC = 2
NS = 16
L = 16
NW = NC * NS
EPW = N_EDGES // NW
G = 80
NCHUNK = EPW // G
K = 25
NP = 10240
DUMMY = N_NODES
RPT = NP // NS


def _sc_scatter_kernel(edge_hbm, x_hbm, acc_out, cnt_out,
                       acc_sh, rowb, colb, cole2d,
                       rows0, rows1, hist, zbuf,
                       sem, sem2, sem3, sem4):
    c = lax.axis_index("c")
    s = lax.axis_index("s")
    wid = c * NS + s

    def _fill_h(i, carry):
        for u in range(4):
            hist[pl.ds((i * 4 + u) * L, L)] = jnp.zeros((L,), jnp.float32)
        return carry
    lax.fori_loop(0, NP // (4 * L), _fill_h, 0)

    def _fill_z(i, carry):
        for j in range(D // L):
            zbuf[i, pl.ds(j * L, L)] = jnp.zeros((L,), jnp.float32)
        return carry
    lax.fori_loop(0, 32, _fill_z, 0)

    base_r = s * RPT
    for k in range(RPT // 32):
        pltpu.sync_copy(zbuf, acc_sh.at[pl.ds(base_r + k * 32, 32)])
    plsc.subcore_barrier()

    ebase = wid * EPW
    rows = (rows0, rows1)
    gsem = (sem, sem2)
    one_v = jnp.ones((L,), jnp.float32)

    def _cole(j):
        for q in range(G // L):
            rv = rowb[pl.ds(j * G + q * L, L)]
            cv = colb[pl.ds(j * G + q * L, L)]
            ce = jnp.where(rv == cv, jnp.int32(DUMMY), cv)
            cole2d[j, pl.ds(q * L, L)] = ce
            plsc.addupdate_scatter(hist, [ce], one_v)

    def _macro(m, carry):
        off = ebase + m * (K * G)
        ri = pltpu.async_copy(edge_hbm.at[0, pl.ds(off, K * G)], rowb, sem)
        ci = pltpu.async_copy(edge_hbm.at[1, pl.ds(off, K * G)], colb, sem2)
        ri.wait()
        ci.wait()
        gd = [None] * K
        ad = [None] * K
        gd[0] = pltpu.async_copy(x_hbm.at[rowb.at[pl.ds(0, G)]], rows[0],
                                 gsem[0])
        _cole(0)
        for j in range(K):
            if j >= 1:
                ad[j - 1].wait()
            if j + 1 < K:
                gd[j + 1] = pltpu.async_copy(
                    x_hbm.at[rowb.at[pl.ds((j + 1) * G, G)]],
                    rows[(j + 1) % 2], gsem[(j + 1) % 2])
                _cole(j + 1)
            gd[j].wait()
            ad[j] = pltpu.async_copy(rows[j % 2], acc_sh.at[cole2d.at[j]],
                                     sem4, add=True)
        ad[K - 1].wait()
        return carry
    lax.fori_loop(0, NCHUNK // K, _macro, 0)

    pltpu.sync_copy(hist, cnt_out.at[wid])
    plsc.subcore_barrier()

    out_base = c * NP + base_r
    pltpu.sync_copy(acc_sh.at[pl.ds(base_r, RPT)],
                    acc_out.at[pl.ds(out_base, RPT)])


_sc_scatter = functools.partial(
    pl.kernel,
    out_type=(
        jax.ShapeDtypeStruct((NC * NP, D), jnp.float32),
        jax.ShapeDtypeStruct((NW, NP), jnp.float32),
    ),
    mesh=plsc.VectorSubcoreMesh(core_axis_name="c", subcore_axis_name="s"),
    scratch_types=[
        pltpu.VMEM_SHARED((NP, D), jnp.float32),
        pltpu.VMEM((K * G,), jnp.int32),
        pltpu.VMEM((K * G,), jnp.int32),
        pltpu.VMEM((K, G), jnp.int32),
        pltpu.VMEM((G, D), jnp.float32),
        pltpu.VMEM((G, D), jnp.float32),
        pltpu.VMEM((NP,), jnp.float32),
        pltpu.VMEM((32, D), jnp.float32),
        pltpu.SemaphoreType.DMA,
        pltpu.SemaphoreType.DMA,
        pltpu.SemaphoreType.DMA,
        pltpu.SemaphoreType.DMA,
    ],
    compiler_params=pltpu.CompilerParams(use_tc_tiling_on_sc=False,
                                         needs_layout_passes=False),
)(_sc_scatter_kernel)


def _tc_finish_kernel(acc_ref, cnt_ref, x_ref, w_ref, b_ref, g_ref, be_ref,
                      o_ref):
    acc = acc_ref[...]
    cnt = cnt_ref[...]
    x = x_ref[...]
    s_tot = acc[0:N_NODES] + acc[NP:NP + N_NODES] + x
    c_tot = (jnp.sum(cnt[:, 0:N_NODES], axis=0) + 1.0).reshape(N_NODES, 1)
    aggr = s_tot / c_tot
    h = lax.dot_general(aggr, w_ref[...], (((1,), (1,)), ((), ())),
                        preferred_element_type=jnp.float32,
                        precision=lax.Precision.HIGHEST)
    h = h + b_ref[...]
    mean = jnp.mean(h, axis=0, keepdims=True)
    var = jnp.mean(jnp.square(h - mean), axis=0, keepdims=True)
    out = (h - mean) * lax.rsqrt(var + 1e-5) * g_ref[...] + be_ref[...] + x
    o_ref[...] = jnp.maximum(out, 0.0)


def _tc_finish(acc, cnt, x, W_lin, b_lin, gamma2, beta2):
    return pl.pallas_call(
        _tc_finish_kernel,
        out_shape=jax.ShapeDtypeStruct((N_NODES, D), jnp.float32),
    )(acc, cnt, x, W_lin, b_lin, gamma2, beta2)


def kernel(x, edge_index, W_lin, b_lin, gamma2, beta2):
    acc, cnt = _sc_scatter(edge_index, x)
    return _tc_finish(acc, cnt, x, W_lin,
                      b_lin.reshape(1, D), gamma2.reshape(1, D),
                      beta2.reshape(1, D))


# speedup vs baseline: 1.1298x; 1.1298x over previous
import functools

import jax
import jax.numpy as jnp
from jax import lax
from jax.experimental import pallas as pl
from jax.experimental.pallas import tpu as pltpu
from jax.experimental.pallas import tpu_sc as plsc

N_NODES = 10000
N_EDGES = 320000
D = 128

NC = 2
NS = 16
L = 16
NW = NC * NS
EPW = N_EDGES // NW
G = 80
NCHUNK = EPW // G
K = 25
NP = 10240
DUMMY = N_NODES
RPT = NP // NS


def _sc_scatter_kernel(edge_hbm, x_hbm, acc_out, cnt_out,
                       acc_sh, rowb, colb, cole2d,
                       rows0, rows1, rows2, hist, zbuf,
                       sem, sem2, sem3, sem4):
    c = lax.axis_index("c")
    s = lax.axis_index("s")
    wid = c * NS + s

    def _fill_h(i, carry):
        for u in range(4):
            hist[pl.ds((i * 4 + u) * L, L)] = jnp.zeros((L,), jnp.float32)
        return carry
    lax.fori_loop(0, NP // (4 * L), _fill_h, 0)

    def _fill_z(i, carry):
        for j in range(D // (2 * L)):
            zbuf[i, pl.ds(j * 2 * L, 2 * L)] = jnp.zeros((2 * L,),
                                                         jnp.bfloat16)
        return carry
    lax.fori_loop(0, 32, _fill_z, 0)

    base_r = s * RPT
    for k in range(RPT // 32):
        pltpu.sync_copy(zbuf, acc_sh.at[pl.ds(base_r + k * 32, 32)])
    plsc.subcore_barrier()

    ebase = wid * EPW
    rows = (rows0, rows1, rows2)
    gsem = (sem, sem2, sem3)
    one_v = jnp.ones((L,), jnp.float32)

    def _cole(j):
        for q in range(G // L):
            rv = rowb[pl.ds(j * G + q * L, L)]
            cv = colb[pl.ds(j * G + q * L, L)]
            ce = jnp.where(rv == cv, jnp.int32(DUMMY), cv)
            cole2d[j, pl.ds(q * L, L)] = ce
            plsc.addupdate_scatter(hist, [ce], one_v)

    def _macro(m, carry):
        off = ebase + m * (K * G)
        ri = pltpu.async_copy(edge_hbm.at[0, pl.ds(off, K * G)], rowb, sem)
        ci = pltpu.async_copy(edge_hbm.at[1, pl.ds(off, K * G)], colb, sem2)
        ri.wait()
        ci.wait()
        gd = [None] * K
        ad = [None] * K
        gd[0] = pltpu.async_copy(x_hbm.at[rowb.at[pl.ds(0, G)]], rows[0],
                                 gsem[0])
        _cole(0)
        for j in range(K):
            if j >= 2:
                ad[j - 2].wait()
            if j + 1 < K:
                gd[j + 1] = pltpu.async_copy(
                    x_hbm.at[rowb.at[pl.ds((j + 1) * G, G)]],
                    rows[(j + 1) % 3], gsem[(j + 1) % 3])
                _cole(j + 1)
            gd[j].wait()
            ad[j] = pltpu.async_copy(rows[j % 3], acc_sh.at[cole2d.at[j]],
                                     sem4, add=True)
        ad[K - 2].wait()
        ad[K - 1].wait()
        return carry
    lax.fori_loop(0, NCHUNK // K, _macro, 0)

    pltpu.sync_copy(hist, cnt_out.at[wid])
    plsc.subcore_barrier()

    out_base = c * NP + base_r
    pltpu.sync_copy(acc_sh.at[pl.ds(base_r, RPT)],
                    acc_out.at[pl.ds(out_base, RPT)])


_sc_scatter = functools.partial(
    pl.kernel,
    out_type=(
        jax.ShapeDtypeStruct((NC * NP, D), jnp.bfloat16),
        jax.ShapeDtypeStruct((NW, NP), jnp.float32),
    ),
    mesh=plsc.VectorSubcoreMesh(core_axis_name="c", subcore_axis_name="s"),
    scratch_types=[
        pltpu.VMEM_SHARED((NP, D), jnp.bfloat16),
        pltpu.VMEM((K * G,), jnp.int32),
        pltpu.VMEM((K * G,), jnp.int32),
        pltpu.VMEM((K, G), jnp.int32),
        pltpu.VMEM((G, D), jnp.bfloat16),
        pltpu.VMEM((G, D), jnp.bfloat16),
        pltpu.VMEM((G, D), jnp.bfloat16),
        pltpu.VMEM((NP,), jnp.float32),
        pltpu.VMEM((32, D), jnp.bfloat16),
        pltpu.SemaphoreType.DMA,
        pltpu.SemaphoreType.DMA,
        pltpu.SemaphoreType.DMA,
        pltpu.SemaphoreType.DMA,
    ],
    compiler_params=pltpu.CompilerParams(use_tc_tiling_on_sc=False,
                                         needs_layout_passes=False),
)(_sc_scatter_kernel)


def _tc_finish_kernel(acc_ref, cnt_ref, x_ref, w_ref, b_ref, g_ref, be_ref,
                      o_ref):
    acc = acc_ref[...]
    cnt = cnt_ref[...]
    x = x_ref[...]
    accf = acc.astype(jnp.float32)
    s_tot = accf[0:N_NODES] + accf[NP:NP + N_NODES] + x
    c_tot = (jnp.sum(cnt[:, 0:N_NODES], axis=0) + 1.0).reshape(N_NODES, 1)
    aggr = s_tot / c_tot
    h = lax.dot_general(aggr, w_ref[...], (((1,), (1,)), ((), ())),
                        preferred_element_type=jnp.float32,
                        precision=lax.Precision.HIGHEST)
    h = h + b_ref[...]
    mean = jnp.mean(h, axis=0, keepdims=True)
    var = jnp.mean(jnp.square(h - mean), axis=0, keepdims=True)
    out = (h - mean) * lax.rsqrt(var + 1e-5) * g_ref[...] + be_ref[...] + x
    o_ref[...] = jnp.maximum(out, 0.0)


def _tc_finish(acc, cnt, x, W_lin, b_lin, gamma2, beta2):
    return pl.pallas_call(
        _tc_finish_kernel,
        out_shape=jax.ShapeDtypeStruct((N_NODES, D), jnp.float32),
    )(acc, cnt, x, W_lin, b_lin, gamma2, beta2)


def kernel(x, edge_index, W_lin, b_lin, gamma2, beta2):
    acc, cnt = _sc_scatter(edge_index, x.astype(jnp.bfloat16))
    return _tc_finish(acc, cnt, x, W_lin,
                      b_lin.reshape(1, D), gamma2.reshape(1, D),
                      beta2.reshape(1, D))
